# baseline (device time: 190360 ns/iter reference)
import jax
import jax.numpy as jnp
from jax import lax
from jax.experimental import pallas as pl
from jax.experimental.pallas import tpu as pltpu

N_DEV = 16
D = 4
N_SUB = 4


def kernel(x, w_mat):
    m, k_sh = x.shape
    _, n = w_mat.shape
    chunk = m // N_DEV
    q = n // (2 * N_SUB)

    def body(x_ref, w_ref, out_ref,
             comm, t0, t1, gather,
             send_sems, recv_sems, cred_sems, aa_send, aa_recv):
        my = lax.axis_index("i")
        left = jnp.mod(my - 1, N_DEV)
        right = jnp.mod(my + 1, N_DEV)

        barrier_sem = pltpu.get_barrier_semaphore()
        for nbr in (left, right):
            pl.semaphore_signal(barrier_sem, inc=1, device_id=(nbr,),
                                device_id_type=pl.DeviceIdType.MESH)
        pl.semaphore_wait(barrier_sem, 2)

        def partial(c, lo):
            return jnp.dot(
                x_ref[pl.ds(c * chunk, chunk), :],
                w_ref[:, lo:lo + q],
                preferred_element_type=jnp.float32,
            )

        class Ring:
            def __init__(self, i):
                self.i = i
                self.cw = i < N_SUB
                self.lo = i * q
                self.target = right if self.cw else left
                self.credit_to = left if self.cw else right
                self.prev_send = None

            def seed_chunk(self):
                return jnp.mod(my - 1, N_DEV) if self.cw else jnp.mod(my + 1, N_DEV)

            def recv_chunk(self, s):
                return (jnp.mod(my - 2 - s, N_DEV) if self.cw
                        else jnp.mod(my + 2 + s, N_DEV))

            def start_send(self, u):
                d = pltpu.make_async_remote_copy(
                    src_ref=comm.at[self.i, u % D],
                    dst_ref=comm.at[self.i, (u + 1) % D],
                    send_sem=send_sems.at[self.i, u % D],
                    recv_sem=recv_sems.at[self.i, (u + 1) % D],
                    device_id=(self.target,),
                    device_id_type=pl.DeviceIdType.MESH,
                )
                d.start()
                self.prev_send = d

            def wait_recv(self, s):
                rs = (s + 1) % D
                d = pltpu.make_async_remote_copy(
                    src_ref=comm.at[self.i, rs],
                    dst_ref=comm.at[self.i, rs],
                    send_sem=send_sems.at[self.i, rs],
                    recv_sem=recv_sems.at[self.i, rs],
                    device_id=(self.target,),
                    device_id_type=pl.DeviceIdType.MESH,
                )
                d.wait_recv()

        rings = [Ring(i) for i in range(2 * N_SUB)]
        groups = [[rings[g], rings[N_SUB + g]] for g in range(N_SUB)]
        tmps = [t0, t1]

        for grp in groups:
            for r in grp:
                comm[r.i, 0, :, :] = partial(r.seed_chunk(), r.lo)
            for r in grp:
                r.start_send(0)

        for s in range(N_DEV - 1):
            rs = (s + 1) % D
            for grp in groups:
                for r, t in zip(grp, tmps):
                    t[...] = partial(r.recv_chunk(s), r.lo)
                for r in grp:
                    r.wait_recv(s)
                    r.prev_send.wait_send()
                if s <= (N_DEV - 2) - (D - 1):
                    for r in grp:
                        pl.semaphore_signal(
                            cred_sems.at[r.i], inc=1,
                            device_id=(r.credit_to,),
                            device_id_type=pl.DeviceIdType.MESH)
                for r, t in zip(grp, tmps):
                    comm[r.i, rs, :, :] = comm[r.i, rs, :, :] + t[...]
                if s < N_DEV - 2:
                    for r in grp:
                        if s + 1 >= D - 1:
                            pl.semaphore_wait(cred_sems.at[r.i], 1)
                        r.start_send(s + 1)

        fin = (N_DEV - 1) % D
        ys = [jnp.maximum(comm[r.i, fin, :, :], 0.0) for r in rings]
        m_own = jnp.max(jnp.stack([jnp.max(y) for y in ys]))

        gather[pl.ds(my, 1)] = jnp.broadcast_to(m_own, (1, 8, 128))
        sends = []
        for k in range(1, N_DEV):
            tgt = jnp.mod(my + k, N_DEV)
            d = pltpu.make_async_remote_copy(
                src_ref=gather.at[my],
                dst_ref=gather.at[my],
                send_sem=aa_send.at[tgt],
                recv_sem=aa_recv.at[my],
                device_id=(tgt,),
                device_id_type=pl.DeviceIdType.MESH,
            )
            d.start()
            sends.append(d)
        for k in range(1, N_DEV):
            src = jnp.mod(my + k, N_DEV)
            d = pltpu.make_async_remote_copy(
                src_ref=gather.at[src],
                dst_ref=gather.at[src],
                send_sem=aa_send.at[src],
                recv_sem=aa_recv.at[src],
                device_id=(src,),
                device_id_type=pl.DeviceIdType.MESH,
            )
            d.wait_recv()
        for d in sends:
            d.wait_send()

        gmax = jnp.max(gather[...])
        scale = gmax / 127.0
        for r, y in zip(rings, ys):
            qv = jnp.clip(jnp.round(y / scale), -127.0, 127.0)
            out_ref[:, r.lo:r.lo + q] = qv * scale

    return pl.pallas_call(
        body,
        out_shape=jax.ShapeDtypeStruct((chunk, n), jnp.float32),
        in_specs=[
            pl.BlockSpec(memory_space=pltpu.VMEM),
            pl.BlockSpec(memory_space=pltpu.VMEM),
        ],
        out_specs=pl.BlockSpec(memory_space=pltpu.VMEM),
        scratch_shapes=[
            pltpu.VMEM((2 * N_SUB, D, chunk, q), jnp.float32),
            pltpu.VMEM((chunk, q), jnp.float32),
            pltpu.VMEM((chunk, q), jnp.float32),
            pltpu.VMEM((N_DEV, 8, 128), jnp.float32),
            pltpu.SemaphoreType.DMA((2 * N_SUB, D)),
            pltpu.SemaphoreType.DMA((2 * N_SUB, D)),
            pltpu.SemaphoreType.REGULAR((2 * N_SUB,)),
            pltpu.SemaphoreType.DMA((N_DEV,)),
            pltpu.SemaphoreType.DMA((N_DEV,)),
        ],
        compiler_params=pltpu.CompilerParams(
            collective_id=0,
            vmem_limit_bytes=100 * 1024 * 1024,
        ),
    )(x, w_mat)
